# Initial kernel scaffold; baseline (speedup 1.0000x reference)
import jax
import jax.numpy as jnp
from jax.experimental import pallas as pl


def kernel(s_feats, neighbor_indices):
    raise NotImplementedError("write your pallas kernel here")


# speedup vs baseline: 4.1501x; 4.1501x over previous
import functools

import jax
import jax.numpy as jnp
from jax import lax
from jax.experimental import pallas as pl
from jax.experimental.pallas import tpu as pltpu
from jax.experimental.pallas import tpu_sc as plsc

N = 10000
D = 128
K = 32
L = 16

NC = 2
NS = 16
NW = NC * NS

R = 320
G = 4
GK = G * K
NCH = R // G

_mesh = plsc.VectorSubcoreMesh(core_axis_name="c", subcore_axis_name="s")


@functools.partial(
    pl.kernel,
    out_type=jax.ShapeDtypeStruct((N, D), jnp.float32),
    mesh=_mesh,
    scratch_types=[
        pltpu.VMEM((R * K,), jnp.int32),
        pltpu.VMEM((GK, D), jnp.float32),
        pltpu.VMEM((GK, D), jnp.float32),
        pltpu.VMEM((R, D), jnp.float32),
        pltpu.SemaphoreType.DMA,
        pltpu.SemaphoreType.DMA,
    ],
)
def _maxpool_sc(feats_hbm, idx_hbm, out_hbm, idx_v, nb0, nb1, out_v, sem0, sem1):
    wid = lax.axis_index("s") * NC + lax.axis_index("c")
    base = jnp.minimum(wid * R, N - R)

    pltpu.sync_copy(idx_hbm.at[pl.ds(base * K, R * K)], idx_v)

    def fire(ch, nb, sem):
        pltpu.make_async_copy(
            feats_hbm.at[idx_v.at[pl.ds(ch * GK, GK)]], nb, sem).start()

    def drain(ch, nb, sem):
        pltpu.make_async_copy(
            feats_hbm.at[idx_v.at[pl.ds(ch * GK, GK)]], nb, sem).wait()

    def reduce_chunk(nb, ch):
        for g in range(G):
            row = ch * G + g
            for c in range(D // L):
                sl = pl.ds(c * L, L)
                acc = nb[g * K, sl]
                for k in range(1, K):
                    acc = jnp.maximum(acc, nb[g * K + k, sl])
                out_v[row, sl] = acc

    fire(0, nb0, sem0)

    @pl.loop(0, NCH, step=2)
    def _(ch):
        fire(ch + 1, nb1, sem1)
        drain(ch, nb0, sem0)
        reduce_chunk(nb0, ch)

        @pl.when(ch + 2 < NCH)
        def _():
            fire(ch + 2, nb0, sem0)

        drain(ch + 1, nb1, sem1)
        reduce_chunk(nb1, ch + 1)

    pltpu.sync_copy(out_v, out_hbm.at[pl.ds(base, R)])


def kernel(s_feats, neighbor_indices):
    idx_flat = neighbor_indices.astype(jnp.int32).reshape(-1)
    return _maxpool_sc(s_feats, idx_flat)
